# Initial kernel scaffold; baseline (speedup 1.0000x reference)
import jax
import jax.numpy as jnp
from jax.experimental import pallas as pl


def kernel(x, edge_index, W1, W2):
    raise NotImplementedError("write your pallas kernel here")


# speedup vs baseline: 3.6971x; 3.6971x over previous
import functools

import jax
import jax.numpy as jnp
from jax import lax
from jax.experimental import pallas as pl
from jax.experimental.pallas import tpu as pltpu
from jax.experimental.pallas import tpu_sc as plsc

N = 10000
D = 128
E = 320000
NW = 32
K = 128
CH = -(-E // (NW * K))
Q = CH * K
EP = NW * Q
R = 10240
ZR = R // 16
OR_ = N // 16
DUMMY = N
DW = 16

_mesh = plsc.VectorSubcoreMesh(core_axis_name="c", subcore_axis_name="s")


def _sc_agg_body(h_hbm, srcp_hbm, dstp_hbm, zeros_hbm, out_hbm,
                 sidx, didx, rows, acc):
    c = lax.axis_index("c")
    s = lax.axis_index("s")
    wid = c * 16 + s
    pltpu.sync_copy(zeros_hbm, acc.at[pl.ds(s * ZR, ZR)])
    plsc.subcore_barrier()
    base = wid * Q

    def body(i, carry):
        off = base + i * K
        pltpu.sync_copy(srcp_hbm.at[pl.ds(off, K)], sidx)
        pltpu.sync_copy(dstp_hbm.at[pl.ds(off, K)], didx)
        pltpu.sync_copy(h_hbm.at[sidx], rows)
        pltpu.sync_copy(rows, acc.at[didx], add=True)
        return carry

    lax.fori_loop(0, CH, body, 0)
    plsc.subcore_barrier()
    pltpu.sync_copy(acc.at[pl.ds(s * ZR, ZR)],
                    out_hbm.at[c, pl.ds(s * ZR, ZR)])


_sc_agg = pl.kernel(
    _sc_agg_body,
    out_type=jax.ShapeDtypeStruct((2, R, D), jnp.float32),
    mesh=_mesh,
    scratch_types=[
        pltpu.VMEM((K,), jnp.int32),
        pltpu.VMEM((K,), jnp.int32),
        pltpu.VMEM((K, D), jnp.float32),
        pltpu.VMEM_SHARED((R, D), jnp.float32),
    ],
)


def _sc_deg_body(dstp_hbm, ones_hbm, zeros_hbm, out_hbm, didx, ones_v, acc):
    c = lax.axis_index("c")
    s = lax.axis_index("s")
    wid = c * 16 + s
    pltpu.sync_copy(zeros_hbm, acc.at[pl.ds(s * ZR, ZR)])
    pltpu.sync_copy(ones_hbm, ones_v)
    plsc.subcore_barrier()
    base = wid * Q

    def body(i, carry):
        off = base + i * K
        pltpu.sync_copy(dstp_hbm.at[pl.ds(off, K)], didx)
        pltpu.sync_copy(ones_v, acc.at[didx], add=True)
        return carry

    lax.fori_loop(0, CH, body, 0)
    plsc.subcore_barrier()
    pltpu.sync_copy(acc.at[pl.ds(s * ZR, ZR)],
                    out_hbm.at[c, pl.ds(s * ZR, ZR)])


_sc_deg = pl.kernel(
    _sc_deg_body,
    out_type=jax.ShapeDtypeStruct((2, R, D), jnp.float32),
    mesh=_mesh,
    scratch_types=[
        pltpu.VMEM((K,), jnp.int32),
        pltpu.VMEM((K, D), jnp.float32),
        pltpu.VMEM_SHARED((R, D), jnp.float32),
    ],
)


def _tc_inv_body(degp_ref, inv_ref):
    d = degp_ref[0, :, 0:1] + degp_ref[1, :, 0:1]
    inv = jnp.where(d > 0, 1.0 / jnp.maximum(d, 1.0), 0.0)
    inv_ref[...] = inv[:N, :]


_tc_inv = pl.pallas_call(
    _tc_inv_body,
    out_shape=jax.ShapeDtypeStruct((N, 1), jnp.float32),
)


def _tc_dense_body(h_ref, parts_ref, inv_ref, w_ref, out_ref):
    a = (parts_ref[0] + parts_ref[1]) * inv_ref[...]
    z = (jnp.dot(h_ref[...], w_ref[:D, :], preferred_element_type=jnp.float32)
         + jnp.dot(a, w_ref[D:, :], preferred_element_type=jnp.float32))
    z = jnp.maximum(z, 0.0)
    nrm = jnp.sqrt(jnp.sum(z * z, axis=1, keepdims=True))
    out_ref[...] = z / jnp.maximum(nrm, 1e-12)


_B = 1000


_tc_dense = pl.pallas_call(
    _tc_dense_body,
    grid=(N // _B,),
    in_specs=[
        pl.BlockSpec((_B, D), lambda i: (i, 0)),
        pl.BlockSpec((2, _B, D), lambda i: (0, i, 0)),
        pl.BlockSpec((_B, 1), lambda i: (i, 0)),
        pl.BlockSpec((2 * D, D), lambda i: (0, 0)),
    ],
    out_specs=pl.BlockSpec((_B, D), lambda i: (i, 0)),
    out_shape=jax.ShapeDtypeStruct((N, D), jnp.float32),
)


def kernel(x, edge_index, W1, W2):
    src = edge_index[0].astype(jnp.int32)
    dst = edge_index[1].astype(jnp.int32)
    pad = EP - E
    srcp = jnp.concatenate([src, jnp.zeros((pad,), jnp.int32)])
    dstp = jnp.concatenate([dst, jnp.full((pad,), DUMMY, jnp.int32)])
    zrows = jnp.zeros((ZR, D), jnp.float32)
    ones = jnp.ones((K, D), jnp.float32)

    degp = _sc_deg(dstp, ones, zrows)
    inv = _tc_inv(degp)
    p1 = _sc_agg(x, srcp, dstp, zrows)
    h1 = _tc_dense(x, p1, inv, W1)
    p2 = _sc_agg(h1, srcp, dstp, zrows)
    h2 = _tc_dense(h1, p2, inv, W2)
    return h2
